# baseline (speedup 1.0000x reference)
import jax
import jax.numpy as jnp
from jax.experimental import pallas as pl

N = 10000; E = 320000; D = 128; H = 128; S = 2; F = 8; FH = 32; OUT = 64; G = 16; C = 10


def _head_kernel(xc_ref, w1_ref, b1_ref, w2_ref, b2_ref, w3_ref, b3_ref, o_ref):
    xc = xc_ref[...]
    o = jnp.maximum(xc @ w1_ref[...] + b1_ref[...], 0.0)
    o = jnp.maximum(o @ w2_ref[...] + b2_ref[...], 0.0)
    o_ref[...] = o @ w3_ref[...] + b3_ref[...]


def kernel(x, edge_index, batch, W_in, b_in, W_ode, b_ode, Wf1, bf1, Wf2, bf2, Wphi, bphi, Wrho, brho, Wr1, br1, Wr2, br2, Wc1, bc1, Wc2, bc2, Wc3, bc3):
    src = edge_index[0]; dst = edge_index[1]
    deg = jax.ops.segment_sum(jnp.ones((E,), jnp.float32), dst, num_segments=N)
    inv_deg = 1.0 / jnp.clip(deg, 1.0)
    h = jnp.tanh(x @ W_in + b_in)
    dt = 1.0 / S
    embeds = []
    for _ in range(S):
        agg = jax.ops.segment_sum(h[src], dst, num_segments=N) * inv_deg[:, None]
        f = jnp.tanh(agg @ W_ode + b_ode)
        h = h + dt * f
        embeds.append(h)
    ode = jnp.stack(embeds)
    ph_vectors = []
    for i in range(S):
        hi = ode[i]
        v = jax.nn.sigmoid(jax.nn.relu(hi @ Wf1[i] + bf1[i]) @ Wf2[i] + bf2[i])
        e = jnp.maximum(v[src], v[dst])
        death = jax.ops.segment_max(e, dst, num_segments=N)
        death = jnp.where(jnp.isfinite(death), jnp.maximum(v, death), v)
        pairs0 = jnp.stack([v, death], axis=-1)
        emb0 = jax.nn.relu(pairs0 @ Wphi[i] + bphi[i])
        pairs1 = jnp.stack([jnp.minimum(v[src], v[dst]), e], axis=-1)
        emb1 = jax.nn.relu(pairs1 @ Wphi[i] + bphi[i])
        g0 = jax.ops.segment_sum(emb0.mean(axis=1), batch, num_segments=G)
        g1 = jax.ops.segment_sum(emb1.mean(axis=1), batch[dst], num_segments=G)
        ph = jnp.tanh((g0 + g1) @ Wrho[i] + brho[i])
        ph_vectors.append(ph)
    ph_embedding = jnp.stack(ph_vectors).mean(axis=0)
    nr = jax.nn.relu(ode[-1] @ Wr1 + br1) @ Wr2 + br2
    xg = jax.ops.segment_sum(nr, batch, num_segments=G)
    xc = jnp.concatenate([xg, ph_embedding], axis=1)
    out = pl.pallas_call(
        _head_kernel,
        out_shape=jax.ShapeDtypeStruct((G, C), jnp.float32),
    )(xc, Wc1, bc1, Wc2, bc2, Wc3, bc3)
    return out


# speedup vs baseline: 1.2620x; 1.2620x over previous
import functools

import jax
import jax.numpy as jnp
from jax import lax
from jax.experimental import pallas as pl
from jax.experimental.pallas import tpu as pltpu
from jax.experimental.pallas import tpu_sc as plsc

N = 10000; E = 320000; D = 128; H = 128; S = 2; F = 8; FH = 32; OUT = 64; G = 16; C = 10

_NW = 32
_EPW = E // _NW
_K = 80
_NCH = _EPW // _K
_RB = 80
_NRB = N // _RB


def _make_seg_sum():
    mesh = plsc.VectorSubcoreMesh(core_axis_name="c", subcore_axis_name="s")

    @functools.partial(
        pl.kernel, mesh=mesh,
        out_type=jax.ShapeDtypeStruct((2, N, H), jnp.float32),
        scratch_types=[
            pltpu.VMEM((_K,), jnp.int32),
            pltpu.VMEM((_K,), jnp.int32),
            pltpu.VMEM((_K, H), jnp.float32),
            pltpu.VMEM((_RB, H), jnp.float32),
            pltpu.VMEM_SHARED((N, H), jnp.float32),
            pltpu.SemaphoreType.DMA,
        ],
    )
    def seg(h_hbm, src_hbm, dst_hbm, z_hbm, out_hbm, sidx, didx, rows, zbuf, acc, sem):
        c = lax.axis_index("c")
        s = lax.axis_index("s")
        wid = s * 2 + c
        pltpu.sync_copy(z_hbm, zbuf)
        def zb(i, carry):
            j = s + i * 16
            @pl.when(j < _NRB)
            def _():
                pltpu.sync_copy(zbuf, acc.at[pl.ds(j * _RB, _RB)])
            return carry
        lax.fori_loop(0, (_NRB + 15) // 16, zb, 0)
        plsc.subcore_barrier()
        base = wid * _EPW
        def body(i, carry):
            off = base + i * _K
            pltpu.sync_copy(src_hbm.at[pl.ds(off, _K)], sidx)
            pltpu.sync_copy(dst_hbm.at[pl.ds(off, _K)], didx)
            pltpu.async_copy(h_hbm.at[sidx], rows, sem).wait()
            pltpu.sync_copy(rows, acc.at[didx], add=True)
            return carry
        lax.fori_loop(0, _NCH, body, 0)
        plsc.subcore_barrier()
        def rb(i, carry):
            j = s + i * 16
            @pl.when(j < _NRB)
            def _():
                r0 = j * _RB
                pltpu.sync_copy(acc.at[pl.ds(r0, _RB)], zbuf)
                pltpu.sync_copy(zbuf, out_hbm.at[c].at[pl.ds(r0, _RB)])
            return carry
        lax.fori_loop(0, (_NRB + 15) // 16, rb, 0)

    return seg


_seg_sum = _make_seg_sum()


def _head_kernel(xc_ref, w1_ref, b1_ref, w2_ref, b2_ref, w3_ref, b3_ref, o_ref):
    xc = xc_ref[...]
    o = jnp.maximum(xc @ w1_ref[...] + b1_ref[...], 0.0)
    o = jnp.maximum(o @ w2_ref[...] + b2_ref[...], 0.0)
    o_ref[...] = o @ w3_ref[...] + b3_ref[...]


def kernel(x, edge_index, batch, W_in, b_in, W_ode, b_ode, Wf1, bf1, Wf2, bf2, Wphi, bphi, Wrho, brho, Wr1, br1, Wr2, br2, Wc1, bc1, Wc2, bc2, Wc3, bc3):
    src = edge_index[0]; dst = edge_index[1]
    deg = jax.ops.segment_sum(jnp.ones((E,), jnp.float32), dst, num_segments=N)
    inv_deg = 1.0 / jnp.clip(deg, 1.0)
    h = jnp.tanh(x @ W_in + b_in)
    dt = 1.0 / S
    zrows = jnp.zeros((_RB, H), jnp.float32)
    embeds = []
    for _ in range(S):
        part = _seg_sum(h, src, dst, zrows)
        agg = (part[0] + part[1]) * inv_deg[:, None]
        f = jnp.tanh(agg @ W_ode + b_ode)
        h = h + dt * f
        embeds.append(h)
    ode = jnp.stack(embeds)
    ph_vectors = []
    for i in range(S):
        hi = ode[i]
        v = jax.nn.sigmoid(jax.nn.relu(hi @ Wf1[i] + bf1[i]) @ Wf2[i] + bf2[i])
        e = jnp.maximum(v[src], v[dst])
        death = jax.ops.segment_max(e, dst, num_segments=N)
        death = jnp.where(jnp.isfinite(death), jnp.maximum(v, death), v)
        pairs0 = jnp.stack([v, death], axis=-1)
        emb0 = jax.nn.relu(pairs0 @ Wphi[i] + bphi[i])
        pairs1 = jnp.stack([jnp.minimum(v[src], v[dst]), e], axis=-1)
        emb1 = jax.nn.relu(pairs1 @ Wphi[i] + bphi[i])
        g0 = jax.ops.segment_sum(emb0.mean(axis=1), batch, num_segments=G)
        g1 = jax.ops.segment_sum(emb1.mean(axis=1), batch[dst], num_segments=G)
        ph = jnp.tanh((g0 + g1) @ Wrho[i] + brho[i])
        ph_vectors.append(ph)
    ph_embedding = jnp.stack(ph_vectors).mean(axis=0)
    nr = jax.nn.relu(ode[-1] @ Wr1 + br1) @ Wr2 + br2
    xg = jax.ops.segment_sum(nr, batch, num_segments=G)
    xc = jnp.concatenate([xg, ph_embedding], axis=1)
    out = pl.pallas_call(
        _head_kernel,
        out_shape=jax.ShapeDtypeStruct((G, C), jnp.float32),
    )(xc, Wc1, bc1, Wc2, bc2, Wc3, bc3)
    return out


# speedup vs baseline: 1.5191x; 1.2038x over previous
import functools

import jax
import jax.numpy as jnp
from jax import lax
from jax.experimental import pallas as pl
from jax.experimental.pallas import tpu as pltpu
from jax.experimental.pallas import tpu_sc as plsc

N = 10000; E = 320000; D = 128; H = 128; S = 2; F = 8; FH = 32; OUT = 64; G = 16; C = 10

_NW = 32
_EPW = E // _NW
_K = 80
_NCH = _EPW // _K
_RB = 80
_NRB = N // _RB


def _make_seg_sum():
    mesh = plsc.VectorSubcoreMesh(core_axis_name="c", subcore_axis_name="s")

    @functools.partial(
        pl.kernel, mesh=mesh,
        out_type=(jax.ShapeDtypeStruct((2, N, H), jnp.float32),
                  jax.ShapeDtypeStruct((2, 80, 128), jnp.float32)),
        scratch_types=[
            pltpu.VMEM((_K,), jnp.int32),
            pltpu.VMEM((_K,), jnp.int32),
            pltpu.VMEM((_K, H), jnp.float32),
            pltpu.VMEM((_RB, H), jnp.float32),
            pltpu.VMEM((_K,), jnp.float32),
            pltpu.VMEM((1, 128), jnp.float32),
            pltpu.VMEM((128,), jnp.float32),
            pltpu.VMEM_SHARED((N, H), jnp.float32),
            pltpu.VMEM_SHARED((80 * 128,), jnp.float32),
            pltpu.SemaphoreType.DMA,
        ],
    )
    def seg(h_hbm, src_hbm, dst_hbm, z_hbm, ones_hbm, z128_hbm, out_hbm, deg_hbm,
            sidx, didx, rows, zbuf, ones_v, dbb, z128, acc, dacc, sem):
        c = lax.axis_index("c")
        s = lax.axis_index("s")
        wid = s * 2 + c
        pltpu.sync_copy(z_hbm, zbuf)
        pltpu.sync_copy(z128_hbm, z128)
        pltpu.sync_copy(ones_hbm, ones_v)
        def zb(i, carry):
            j = s + i * 16
            @pl.when(j < _NRB)
            def _():
                pltpu.sync_copy(zbuf, acc.at[pl.ds(j * _RB, _RB)])
            return carry
        lax.fori_loop(0, (_NRB + 15) // 16, zb, 0)
        def zd(i, carry):
            j = s + i * 16
            pltpu.sync_copy(z128, dacc.at[pl.ds(j * 128, 128)])
            return carry
        lax.fori_loop(0, 5, zd, 0)
        plsc.subcore_barrier()
        base = wid * _EPW
        def body(i, carry):
            off = base + i * _K
            pltpu.sync_copy(src_hbm.at[pl.ds(off, _K)], sidx)
            pltpu.sync_copy(dst_hbm.at[pl.ds(off, _K)], didx)
            pltpu.async_copy(h_hbm.at[sidx], rows, sem).wait()
            pltpu.sync_copy(rows, acc.at[didx], add=True)
            pltpu.sync_copy(ones_v, dacc.at[didx], add=True)
            return carry
        lax.fori_loop(0, _NCH, body, 0)
        plsc.subcore_barrier()
        def rb(i, carry):
            j = s + i * 16
            @pl.when(j < _NRB)
            def _():
                r0 = j * _RB
                pltpu.sync_copy(acc.at[pl.ds(r0, _RB)], zbuf)
                pltpu.sync_copy(zbuf, out_hbm.at[c].at[pl.ds(r0, _RB)])
            return carry
        lax.fori_loop(0, (_NRB + 15) // 16, rb, 0)
        def rd(i, carry):
            j = s + i * 16
            pltpu.sync_copy(dacc.at[pl.ds(j * 128, 128)], dbb.at[0])
            pltpu.sync_copy(dbb, deg_hbm.at[c].at[pl.ds(j, 1)])
            return carry
        lax.fori_loop(0, 5, rd, 0)

    return seg


_seg_sum = _make_seg_sum()


_EB = 2560
_NBLK = 2000


def _ph_pool_body(a_ref, b_ref, idx_ref, w_ref, bias_ref, o_ref):
    i = pl.program_id(0)
    t = jnp.zeros((a_ref.shape[0], OUT), jnp.float32)
    w0 = w_ref[0:1, :]
    w1 = w_ref[1:2, :]
    bias = bias_ref[...]
    for f in range(F):
        t += jnp.maximum(a_ref[:, f:f + 1] * w0 + b_ref[:, f:f + 1] * w1 + bias, 0.0)
    t = t * (1.0 / F)
    idx = idx_ref[0, 0, :]
    oh = (jax.lax.broadcasted_iota(jnp.int32, (G, a_ref.shape[0]), 0)
          == idx[None, :]).astype(jnp.float32)
    part = jax.lax.dot(oh, t, preferred_element_type=jnp.float32)
    @pl.when(i == 0)
    def _():
        o_ref[...] = jnp.zeros_like(o_ref)
    o_ref[...] += part


def _ph_pool(a, b, idx3, w, bias, rows, blk):
    nb = rows // blk
    return pl.pallas_call(
        _ph_pool_body,
        grid=(nb,),
        in_specs=[
            pl.BlockSpec((blk, F), lambda i: (i, 0)),
            pl.BlockSpec((blk, F), lambda i: (i, 0)),
            pl.BlockSpec((1, 1, blk), lambda i: (i, 0, 0)),
            pl.BlockSpec((2, OUT), lambda i: (0, 0)),
            pl.BlockSpec((1, OUT), lambda i: (0, 0)),
        ],
        out_specs=pl.BlockSpec((G, OUT), lambda i: (0, 0)),
        out_shape=jax.ShapeDtypeStruct((G, OUT), jnp.float32),
    )(a, b, idx3, w, bias)


def _sum_pool_body(a_ref, idx_ref, o_ref):
    i = pl.program_id(0)
    idx = idx_ref[0, 0, :]
    oh = (jax.lax.broadcasted_iota(jnp.int32, (G, a_ref.shape[0]), 0)
          == idx[None, :]).astype(jnp.float32)
    part = jax.lax.dot(oh, a_ref[...], preferred_element_type=jnp.float32)
    @pl.when(i == 0)
    def _():
        o_ref[...] = jnp.zeros_like(o_ref)
    o_ref[...] += part


def _sum_pool(a, idx3, rows, blk, k):
    nb = rows // blk
    return pl.pallas_call(
        _sum_pool_body,
        grid=(nb,),
        in_specs=[
            pl.BlockSpec((blk, k), lambda i: (i, 0)),
            pl.BlockSpec((1, 1, blk), lambda i: (i, 0, 0)),
        ],
        out_specs=pl.BlockSpec((G, k), lambda i: (0, 0)),
        out_shape=jax.ShapeDtypeStruct((G, k), jnp.float32),
    )(a, idx3)


def _head_kernel(xc_ref, w1_ref, b1_ref, w2_ref, b2_ref, w3_ref, b3_ref, o_ref):
    xc = xc_ref[...]
    o = jnp.maximum(xc @ w1_ref[...] + b1_ref[...], 0.0)
    o = jnp.maximum(o @ w2_ref[...] + b2_ref[...], 0.0)
    o_ref[...] = o @ w3_ref[...] + b3_ref[...]


def kernel(x, edge_index, batch, W_in, b_in, W_ode, b_ode, Wf1, bf1, Wf2, bf2, Wphi, bphi, Wrho, brho, Wr1, br1, Wr2, br2, Wc1, bc1, Wc2, bc2, Wc3, bc3):
    src = edge_index[0]; dst = edge_index[1]
    batch3 = batch.reshape(N // _NBLK, 1, _NBLK)
    bd3 = batch[dst].reshape(E // _EB, 1, _EB)
    h = jnp.tanh(x @ W_in + b_in)
    dt = 1.0 / S
    zrows = jnp.zeros((_RB, H), jnp.float32)
    ones_k = jnp.ones((_K,), jnp.float32)
    zeros128 = jnp.zeros((128,), jnp.float32)
    inv_deg = None
    embeds = []
    for _ in range(S):
        part, dpart = _seg_sum(h, src, dst, zrows, ones_k, zeros128)
        if inv_deg is None:
            deg = (dpart[0] + dpart[1]).reshape(-1)[:N]
            inv_deg = 1.0 / jnp.clip(deg, 1.0)
        agg = (part[0] + part[1]) * inv_deg[:, None]
        f = jnp.tanh(agg @ W_ode + b_ode)
        h = h + dt * f
        embeds.append(h)
    ode = jnp.stack(embeds)
    ph_vectors = []
    for i in range(S):
        hi = ode[i]
        v = jax.nn.sigmoid(jax.nn.relu(hi @ Wf1[i] + bf1[i]) @ Wf2[i] + bf2[i])
        vsrc = v[src]; vdst = v[dst]
        e = jnp.maximum(vsrc, vdst)
        death = jax.ops.segment_max(e, dst, num_segments=N)
        death = jnp.where(jnp.isfinite(death), jnp.maximum(v, death), v)
        bphi_i = bphi[i].reshape(1, OUT)
        g0 = _ph_pool(v, death, batch3, Wphi[i], bphi_i, N, _NBLK)
        g1 = _ph_pool(jnp.minimum(vsrc, vdst), e, bd3, Wphi[i], bphi_i, E, _EB)
        ph = jnp.tanh((g0 + g1) @ Wrho[i] + brho[i])
        ph_vectors.append(ph)
    ph_embedding = jnp.stack(ph_vectors).mean(axis=0)
    nr = jax.nn.relu(ode[-1] @ Wr1 + br1) @ Wr2 + br2
    xg = _sum_pool(nr, batch3, N, _NBLK, H)
    xc = jnp.concatenate([xg, ph_embedding], axis=1)
    out = pl.pallas_call(
        _head_kernel,
        out_shape=jax.ShapeDtypeStruct((G, C), jnp.float32),
    )(xc, Wc1, bc1, Wc2, bc2, Wc3, bc3)
    return out
